# Initial kernel scaffold; baseline (speedup 1.0000x reference)
import jax
import jax.numpy as jnp
from jax.experimental import pallas as pl


def kernel(x, table, mat, mat1):
    raise NotImplementedError("write your pallas kernel here")


# speedup vs baseline: 11.4518x; 11.4518x over previous
import functools

import jax
import jax.numpy as jnp
from jax import lax
from jax.experimental import pallas as pl
from jax.experimental.pallas import tpu as pltpu
from jax.experimental.pallas import tpu_sc as plsc

_NC = 2
_NS = 16
_NW = _NC * _NS

_T = 128
_TPC = 20
_CHUNK = _T * _TPC


def _sc_gather_body(n_chunks, x_hbm, table_hbm, out_hbm, idx_v, rows_v, sem):
    wid = lax.axis_index("s") * _NC + lax.axis_index("c")
    base = wid * (n_chunks * _CHUNK)

    def chunk_body(g, carry):
        off = base + g * _CHUNK
        pltpu.sync_copy(x_hbm.at[pl.ds(off, _CHUNK)], idx_v)
        copies = [
            pltpu.async_copy(
                table_hbm.at[idx_v.at[pl.ds(j * _T, _T)]],
                rows_v.at[pl.ds(j * _T, _T)],
                sem,
            )
            for j in range(_TPC)
        ]
        for c in copies:
            c.wait()
        pltpu.sync_copy(rows_v, out_hbm.at[pl.ds(off, _CHUNK)])
        return carry

    lax.fori_loop(0, n_chunks, chunk_body, 0)


def _sc_gather(x_flat, table):
    n = x_flat.shape[0]
    d = table.shape[1]
    assert n % (_NW * _CHUNK) == 0
    n_chunks = n // (_NW * _CHUNK)
    mesh = plsc.VectorSubcoreMesh(
        core_axis_name="c", subcore_axis_name="s",
        num_cores=_NC, num_subcores=_NS,
    )
    kern = pl.kernel(
        functools.partial(_sc_gather_body, n_chunks),
        out_type=jax.ShapeDtypeStruct((n, d), jnp.float32),
        mesh=mesh,
        scratch_types=[
            pltpu.VMEM((_CHUNK,), jnp.int32),
            pltpu.VMEM((_CHUNK, d), jnp.float32),
            pltpu.SemaphoreType.DMA,
        ],
        compiler_params=pltpu.CompilerParams(use_tc_tiling_on_sc=False),
    )
    return kern(x_flat, table)


def _mm_body(emb_ref, m_ref, m1_ref, o1_ref, o2_ref):
    e = emb_ref[...]
    o1_ref[...] = jax.lax.dot_general(
        e, m_ref[...], (((1,), (1,)), ((), ())),
        preferred_element_type=jnp.float32,
        precision=jax.lax.Precision.HIGHEST,
    )
    o2_ref[...] = jax.lax.dot_general(
        e, m1_ref[...], (((1,), (1,)), ((), ())),
        preferred_element_type=jnp.float32,
        precision=jax.lax.Precision.HIGHEST,
    )


def _tc_project(emb, mat, mat1):
    n, d = emb.shape
    o = mat.shape[0]
    blk = 8192
    assert n % blk == 0
    grid = (n // blk,)
    out1, out2 = pl.pallas_call(
        _mm_body,
        grid=grid,
        in_specs=[
            pl.BlockSpec((blk, d), lambda i: (i, 0)),
            pl.BlockSpec((o, d), lambda i: (0, 0)),
            pl.BlockSpec((o, d), lambda i: (0, 0)),
        ],
        out_specs=[
            pl.BlockSpec((blk, o), lambda i: (i, 0)),
            pl.BlockSpec((blk, o), lambda i: (i, 0)),
        ],
        out_shape=[
            jax.ShapeDtypeStruct((n, o), jnp.float32),
            jax.ShapeDtypeStruct((n, o), jnp.float32),
        ],
        compiler_params=pltpu.CompilerParams(
            dimension_semantics=("arbitrary",),
        ),
    )(emb, mat, mat1)
    return out1, out2


def kernel(x, table, mat, mat1):
    batch, length = x.shape
    x_flat = x.reshape(-1)
    emb = _sc_gather(x_flat, table)
    out1, out2 = _tc_project(emb, mat, mat1)
    o = mat.shape[0]
    return (out1.reshape(batch, length, o), out2.reshape(batch, length, o))
